# baseline (speedup 1.0000x reference)
import functools

import jax
import jax.numpy as jnp
from jax import lax
from jax.experimental import pallas as pl
from jax.experimental.pallas import tpu as pltpu
from jax.experimental.pallas import tpu_sc as plsc

N = 10000
NPAD = 10240
E = 320000
EPAD = 327680
NUM_GRAPHS = 64

_NSUB = 16
_C = 128


def _make_agg(dh, edge_split):
    nworker = 2 * _NSUB if edge_split else _NSUB
    per_w = EPAD // nworker
    nch = per_w // _C
    hb = 40
    assert hb % 2 == 0 and nch % hb == 0
    nhalves = nch // hb
    nrow = NPAD // _NSUB
    mesh = plsc.VectorSubcoreMesh(core_axis_name="c", subcore_axis_name="s")

    scratch = [
        pltpu.VMEM_SHARED((NPAD, dh), jnp.float32),
        pltpu.VMEM((hb, _C), jnp.int32),
        pltpu.VMEM((hb, _C), jnp.int32),
        pltpu.VMEM((hb, _C), jnp.float32),
        pltpu.VMEM((_C, dh), jnp.float32),
        pltpu.VMEM((_C, dh), jnp.float32),
        pltpu.SemaphoreType.DMA,
        pltpu.SemaphoreType.DMA,
        pltpu.SemaphoreType.DMA,
        pltpu.SemaphoreType.DMA,
        pltpu.SemaphoreType.DMA,
    ]

    @functools.partial(
        pl.kernel,
        out_type=jax.ShapeDtypeStruct((2 * NPAD, dh), jnp.float32),
        mesh=mesh,
        scratch_types=scratch,
    )
    def agg_kernel(xh, src2d, dst2d, ew2d, y, acc, idxs, idxd, wv,
                   r0, r1, sem_i, sg0, sg1, ss0, ss1):
        rbuf = (r0, r1)
        sg = (sg0, sg1)
        ss = (ss0, ss1)
        cid = lax.axis_index("c")
        sid = lax.axis_index("s")
        row0 = sid * nrow
        xoff = 0 if edge_split else cid * NPAD
        wid = cid * _NSUB + sid if edge_split else sid
        ch0 = wid * nch

        def fill(half):
            b0 = ch0 + half * hb
            di = pltpu.async_copy(src2d.at[pl.ds(b0, hb)], idxs, sem_i)
            dd = pltpu.async_copy(dst2d.at[pl.ds(b0, hb)], idxd, sem_i)
            dw = pltpu.async_copy(ew2d.at[pl.ds(b0, hb)], wv, sem_i)
            return di, dd, dw

        def fill_wait(descs):
            for d in descs:
                d.wait()
            if not edge_split:
                def adj_body(j, carry):
                    for g in range(_C // 16):
                        sl = pl.ds(g * 16, 16)
                        idxs[j, sl] = idxs[j, sl] + xoff
                    return carry

                lax.fori_loop(0, hb, adj_body, 0)

        descs = fill(0)
        pltpu.sync_copy(xh.at[pl.ds(xoff + row0, nrow)], acc.at[pl.ds(row0, nrow)])
        fill_wait(descs)
        plsc.subcore_barrier()

        def g_issue(c, k):
            pltpu.async_copy(xh.at[idxs.at[c]], rbuf[k], sg[k])

        def g_wait(c, k):
            pltpu.make_async_copy(xh.at[idxs.at[c]], rbuf[k], sg[k]).wait()

        def s_issue(c, k):
            pltpu.async_copy(rbuf[k], acc.at[idxd.at[c]], ss[k], add=True)

        def s_wait(c, k):
            pltpu.make_async_copy(rbuf[k], acc.at[idxd.at[c]], ss[k]).wait()

        def mul(c, k):
            r = rbuf[k]

            def grp(g, carry):
                w16 = wv[c, pl.ds(g * 16, 16)]
                for e in range(16):
                    w = w16[e]
                    row = g * 16 + e
                    for cc in range(dh // 16):
                        sl = pl.ds(cc * 16, 16)
                        r[row, sl] = r[row, sl] * w
                return carry

            lax.fori_loop(0, _C // 16, grp, 0)

        for half in range(nhalves):
            if half > 0:
                fill_wait(fill(half))
            g_issue(0, 0)
            g_issue(1, 1)

            def pair(i, carry):
                c0 = 2 * i
                c1 = c0 + 1
                g_wait(c0, 0)
                mul(c0, 0)
                s_issue(c0, 0)
                g_wait(c1, 1)
                mul(c1, 1)
                s_issue(c1, 1)

                @pl.when(c0 + 2 < hb)
                def _():
                    s_wait(c0, 0)
                    g_issue(c0 + 2, 0)

                @pl.when(c1 + 2 < hb)
                def _():
                    s_wait(c1, 1)
                    g_issue(c1 + 2, 1)

                return carry

            lax.fori_loop(0, hb // 2, pair, 0)
            s_wait(hb - 2, 0)
            s_wait(hb - 1, 1)

        plsc.subcore_barrier()
        pltpu.sync_copy(acc.at[pl.ds(row0, nrow)], y.at[pl.ds(cid * NPAD + row0, nrow)])

    return agg_kernel


def _mlp_body(mode, refs):
    if mode == 0:
        (p0, p1, p2, w1_ref, b1_ref, w2_ref, b2_ref, g_ref, b_ref, batch_ref,
         out_ref, stk_ref, pool_ref) = refs
        h = p0[...] + p1[...] - p2[...]
        pre = jnp.dot(h, w1_ref[...], preferred_element_type=jnp.float32)
    else:
        (p0, p1, w1a_ref, w1b_ref, b1_ref, w2_ref, b2_ref, g_ref, b_ref,
         batch_ref, out_ref, stk_ref, pool_ref) = refs
        pre = (jnp.dot(p0[...], w1a_ref[...], preferred_element_type=jnp.float32)
               + jnp.dot(p1[...], w1b_ref[...], preferred_element_type=jnp.float32))
    hid = jnp.maximum(pre + b1_ref[...], 0.0)
    z = jnp.dot(hid, w2_ref[...], preferred_element_type=jnp.float32) + b2_ref[...]
    z = jnp.maximum(z, 0.0)
    mean = jnp.mean(z, axis=0, keepdims=True)
    var = jnp.mean((z - mean) * (z - mean), axis=0, keepdims=True)
    zn = (z - mean) * lax.rsqrt(var + 1e-5) * g_ref[...] + b_ref[...]
    out_ref[...] = zn
    if stk_ref is not None:
        dim = zn.shape[1]
        stk_ref[0, pl.ds(0, N), :] = zn[:, : dim // 2]
        stk_ref[1, pl.ds(0, N), :] = zn[:, dim // 2 :]
    seg = jax.lax.broadcasted_iota(jnp.int32, (NUM_GRAPHS, N), 0)
    onehot = (seg == batch_ref[...]).astype(jnp.float32)
    pool_ref[...] = jnp.dot(onehot, zn, preferred_element_type=jnp.float32)


def _mlp(mode, parts, w1, b1, w2, b2, g, b, batch2d, want_stacked):
    dim = w1.shape[1]
    out_shapes = [
        jax.ShapeDtypeStruct((N, dim), jnp.float32),
        jax.ShapeDtypeStruct((2, NPAD, dim // 2), jnp.float32),
        jax.ShapeDtypeStruct((NUM_GRAPHS, dim), jnp.float32),
    ]
    if not want_stacked:
        out_shapes.pop(1)

    def wrapped(*refs):
        if want_stacked:
            _mlp_body(mode, refs)
        else:
            rs = list(refs)
            rs.insert(len(rs) - 1, None)
            _mlp_body(mode, rs)

    if mode == 0:
        wargs = (w1, b1.reshape(1, dim), w2, b2.reshape(1, dim))
    else:
        din = w1.shape[0]
        wargs = (w1[: din // 2], w1[din // 2 :], b1.reshape(1, dim), w2,
                 b2.reshape(1, dim))
    outs = pl.pallas_call(wrapped, out_shape=tuple(out_shapes))(
        *parts, *wargs, g.reshape(1, dim), b.reshape(1, dim), batch2d)
    if want_stacked:
        zn, stk, pool = outs
        return zn, stk.reshape(2 * NPAD, dim // 2), pool
    zn, pool = outs
    return zn, None, pool


def _pad_edges(a):
    pad = EPAD - E
    if a.dtype == jnp.int32:
        fill = (jnp.arange(pad, dtype=jnp.int32) * 37) % N
    else:
        fill = jnp.zeros((pad,), dtype=a.dtype)
    return jnp.concatenate([a, fill]).reshape(EPAD // _C, _C)


def kernel(x, edge_index, edge_weight, batch,
           l0_w1, l0_b1, l0_w2, l0_b2,
           l1_w1, l1_b1, l1_w2, l1_b2,
           l2_w1, l2_b1, l2_w2, l2_b2,
           bn0_g, bn0_b, bn1_g, bn1_b, bn2_g, bn2_b):
    src2d = _pad_edges(edge_index[0])
    dst2d = _pad_edges(edge_index[1])
    ew2d = _pad_edges(edge_weight)
    batch2d = batch.reshape(1, N)
    layers = [
        (l0_w1, l0_b1, l0_w2, l0_b2, bn0_g, bn0_b),
        (l1_w1, l1_b1, l1_w2, l1_b2, bn1_g, bn1_b),
        (l2_w1, l2_b1, l2_w2, l2_b2, bn2_g, bn2_b),
    ]
    hs, pools = [], []
    stacked = None
    for li, (w1, b1, w2, b2, g, b) in enumerate(layers):
        if li == 0:
            agg_fn = _make_agg(x.shape[1], True)
            xp = jnp.pad(x, ((0, NPAD - N), (0, 0)))
            y = agg_fn(xp, src2d, dst2d, ew2d)
            parts = (y[:N], y[NPAD : NPAD + N], x)
            mode = 0
        else:
            agg_fn = _make_agg(128, False)
            y = agg_fn(stacked, src2d, dst2d, ew2d)
            parts = (y[:N], y[NPAD : NPAD + N])
            mode = 1
        h, stacked, p = _mlp(mode, parts, w1, b1, w2, b2, g, b, batch2d,
                             want_stacked=(li < 2))
        hs.append(h)
        pools.append(p)
    x_nodes = jnp.concatenate(hs, axis=1)
    x_g = jnp.concatenate(pools, axis=1)
    return (x_g, x_nodes)


# speedup vs baseline: 1.3287x; 1.3287x over previous
import functools

import jax
import jax.numpy as jnp
from jax import lax
from jax.experimental import pallas as pl
from jax.experimental.pallas import tpu as pltpu
from jax.experimental.pallas import tpu_sc as plsc

N = 10000
NPAD = 10240
E = 320000
EPAD = 327680
NUM_GRAPHS = 64

_NSUB = 16
_C = 128


def _make_agg(dh, edge_split):
    nworker = 2 * _NSUB if edge_split else _NSUB
    per_w = EPAD // nworker
    nch = per_w // _C
    hb = 40
    assert hb % 2 == 0 and nch % hb == 0
    nhalves = nch // hb
    nrow = NPAD // _NSUB
    mesh = plsc.VectorSubcoreMesh(core_axis_name="c", subcore_axis_name="s")

    scratch = [
        pltpu.VMEM_SHARED((NPAD, dh), jnp.float32),
        pltpu.VMEM((hb, _C), jnp.int32),
        pltpu.VMEM((hb, _C), jnp.int32),
        pltpu.VMEM((hb, _C), jnp.float32),
        pltpu.VMEM((_C, dh), jnp.float32),
        pltpu.VMEM((_C, dh), jnp.float32),
        pltpu.SemaphoreType.DMA,
        pltpu.SemaphoreType.DMA,
        pltpu.SemaphoreType.DMA,
        pltpu.SemaphoreType.DMA,
        pltpu.SemaphoreType.DMA,
    ]

    @functools.partial(
        pl.kernel,
        out_type=jax.ShapeDtypeStruct((2 * NPAD, dh), jnp.float32),
        mesh=mesh,
        scratch_types=scratch,
    )
    def agg_kernel(xh, src2d, dst2d, ew2d, y, acc, idxs, idxd, wv,
                   r0, r1, sem_i, sg0, sg1, ss0, ss1):
        rbuf = (r0, r1)
        sg = (sg0, sg1)
        ss = (ss0, ss1)
        cid = lax.axis_index("c")
        sid = lax.axis_index("s")
        row0 = sid * nrow
        xoff = 0 if edge_split else cid * NPAD
        wid = cid * _NSUB + sid if edge_split else sid
        ch0 = wid * nch

        def fill(half):
            b0 = ch0 + half * hb
            di = pltpu.async_copy(src2d.at[pl.ds(b0, hb)], idxs, sem_i)
            dd = pltpu.async_copy(dst2d.at[pl.ds(b0, hb)], idxd, sem_i)
            dw = pltpu.async_copy(ew2d.at[pl.ds(b0, hb)], wv, sem_i)
            return di, dd, dw

        def fill_wait(descs):
            for d in descs:
                d.wait()
            if not edge_split:
                def adj_body(j, carry):
                    for g in range(_C // 16):
                        sl = pl.ds(g * 16, 16)
                        idxs[j, sl] = idxs[j, sl] + xoff
                    return carry

                lax.fori_loop(0, hb, adj_body, 0)

        descs = fill(0)
        pltpu.sync_copy(xh.at[pl.ds(xoff + row0, nrow)], acc.at[pl.ds(row0, nrow)])
        fill_wait(descs)
        plsc.subcore_barrier()

        def g_issue(c, k):
            pltpu.async_copy(xh.at[idxs.at[c]], rbuf[k], sg[k])

        def g_wait(c, k):
            pltpu.make_async_copy(xh.at[idxs.at[c]], rbuf[k], sg[k]).wait()

        def s_issue(c, k):
            pltpu.async_copy(rbuf[k], acc.at[idxd.at[c]], ss[k], add=True)

        def s_wait(c, k):
            pltpu.make_async_copy(rbuf[k], acc.at[idxd.at[c]], ss[k]).wait()

        def mul(c, k):
            r = rbuf[k]

            @functools.partial(plsc.parallel_loop, 0, _C // 16, unroll=2)
            def _(g):
                w16 = wv[c, pl.ds(g * 16, 16)]
                for e in range(16):
                    w = w16[e]
                    row = g * 16 + e
                    for cc in range(dh // 16):
                        sl = pl.ds(cc * 16, 16)
                        r[row, sl] = r[row, sl] * w

        def mul_scatter(c, k):
            mul(c, k)
            pltpu.sync_copy(rbuf[k], acc.at[idxd.at[c]], add=True)

        for half in range(nhalves):
            if half > 0:
                fill_wait(fill(half))
            g_issue(0, 0)
            g_issue(1, 1)

            def pair(i, carry):
                c0 = 2 * i
                c1 = c0 + 1
                g_wait(c0, 0)
                mul_scatter(c0, 0)

                @pl.when(c0 + 2 < hb)
                def _():
                    g_issue(c0 + 2, 0)

                g_wait(c1, 1)
                mul_scatter(c1, 1)

                @pl.when(c1 + 2 < hb)
                def _():
                    g_issue(c1 + 2, 1)

                return carry

            lax.fori_loop(0, hb // 2, pair, 0)

        plsc.subcore_barrier()
        pltpu.sync_copy(acc.at[pl.ds(row0, nrow)], y.at[pl.ds(cid * NPAD + row0, nrow)])

    return agg_kernel


def _mlp_body(mode, refs):
    if mode == 0:
        (p0, p1, p2, w1_ref, b1_ref, w2_ref, b2_ref, g_ref, b_ref, batch_ref,
         out_ref, stk_ref, pool_ref) = refs
        h = p0[...] + p1[...] - p2[...]
        pre = jnp.dot(h, w1_ref[...], preferred_element_type=jnp.float32)
    else:
        (p0, p1, w1a_ref, w1b_ref, b1_ref, w2_ref, b2_ref, g_ref, b_ref,
         batch_ref, out_ref, stk_ref, pool_ref) = refs
        pre = (jnp.dot(p0[...], w1a_ref[...], preferred_element_type=jnp.float32)
               + jnp.dot(p1[...], w1b_ref[...], preferred_element_type=jnp.float32))
    hid = jnp.maximum(pre + b1_ref[...], 0.0)
    z = jnp.dot(hid, w2_ref[...], preferred_element_type=jnp.float32) + b2_ref[...]
    z = jnp.maximum(z, 0.0)
    mean = jnp.mean(z, axis=0, keepdims=True)
    var = jnp.mean((z - mean) * (z - mean), axis=0, keepdims=True)
    zn = (z - mean) * lax.rsqrt(var + 1e-5) * g_ref[...] + b_ref[...]
    out_ref[...] = zn
    if stk_ref is not None:
        dim = zn.shape[1]
        stk_ref[0, pl.ds(0, N), :] = zn[:, : dim // 2]
        stk_ref[1, pl.ds(0, N), :] = zn[:, dim // 2 :]
    seg = jax.lax.broadcasted_iota(jnp.int32, (NUM_GRAPHS, N), 0)
    onehot = (seg == batch_ref[...]).astype(jnp.float32)
    pool_ref[...] = jnp.dot(onehot, zn, preferred_element_type=jnp.float32)


def _mlp(mode, parts, w1, b1, w2, b2, g, b, batch2d, want_stacked):
    dim = w1.shape[1]
    out_shapes = [
        jax.ShapeDtypeStruct((N, dim), jnp.float32),
        jax.ShapeDtypeStruct((2, NPAD, dim // 2), jnp.float32),
        jax.ShapeDtypeStruct((NUM_GRAPHS, dim), jnp.float32),
    ]
    if not want_stacked:
        out_shapes.pop(1)

    def wrapped(*refs):
        if want_stacked:
            _mlp_body(mode, refs)
        else:
            rs = list(refs)
            rs.insert(len(rs) - 1, None)
            _mlp_body(mode, rs)

    if mode == 0:
        wargs = (w1, b1.reshape(1, dim), w2, b2.reshape(1, dim))
    else:
        din = w1.shape[0]
        wargs = (w1[: din // 2], w1[din // 2 :], b1.reshape(1, dim), w2,
                 b2.reshape(1, dim))
    outs = pl.pallas_call(wrapped, out_shape=tuple(out_shapes))(
        *parts, *wargs, g.reshape(1, dim), b.reshape(1, dim), batch2d)
    if want_stacked:
        zn, stk, pool = outs
        return zn, stk.reshape(2 * NPAD, dim // 2), pool
    zn, pool = outs
    return zn, None, pool


def _pad_edges(a):
    pad = EPAD - E
    if a.dtype == jnp.int32:
        fill = (jnp.arange(pad, dtype=jnp.int32) * 37) % N
    else:
        fill = jnp.zeros((pad,), dtype=a.dtype)
    return jnp.concatenate([a, fill]).reshape(EPAD // _C, _C)


def kernel(x, edge_index, edge_weight, batch,
           l0_w1, l0_b1, l0_w2, l0_b2,
           l1_w1, l1_b1, l1_w2, l1_b2,
           l2_w1, l2_b1, l2_w2, l2_b2,
           bn0_g, bn0_b, bn1_g, bn1_b, bn2_g, bn2_b):
    src2d = _pad_edges(edge_index[0])
    dst2d = _pad_edges(edge_index[1])
    ew2d = _pad_edges(edge_weight)
    batch2d = batch.reshape(1, N)
    layers = [
        (l0_w1, l0_b1, l0_w2, l0_b2, bn0_g, bn0_b),
        (l1_w1, l1_b1, l1_w2, l1_b2, bn1_g, bn1_b),
        (l2_w1, l2_b1, l2_w2, l2_b2, bn2_g, bn2_b),
    ]
    hs, pools = [], []
    stacked = None
    for li, (w1, b1, w2, b2, g, b) in enumerate(layers):
        if li == 0:
            agg_fn = _make_agg(x.shape[1], True)
            xp = jnp.pad(x, ((0, NPAD - N), (0, 0)))
            y = agg_fn(xp, src2d, dst2d, ew2d)
            parts = (y[:N], y[NPAD : NPAD + N], x)
            mode = 0
        else:
            agg_fn = _make_agg(128, False)
            y = agg_fn(stacked, src2d, dst2d, ew2d)
            parts = (y[:N], y[NPAD : NPAD + N])
            mode = 1
        h, stacked, p = _mlp(mode, parts, w1, b1, w2, b2, g, b, batch2d,
                             want_stacked=(li < 2))
        hs.append(h)
        pools.append(p)
    x_nodes = jnp.concatenate(hs, axis=1)
    x_g = jnp.concatenate(pools, axis=1)
    return (x_g, x_nodes)
